# Initial kernel scaffold; baseline (speedup 1.0000x reference)
import jax
import jax.numpy as jnp
from jax.experimental import pallas as pl


def kernel(x, edge_index, edge_attr, batchs, energy, Wq, bq, Wk, bk, Wv, bv, We, be, Ws, bs, W_fc0, b_fc0, W_lin1, b_lin1, W_fc1, b_fc1, W_out, b_out):
    raise NotImplementedError("write your pallas kernel here")


# speedup vs baseline: 2.1301x; 2.1301x over previous
import functools
import math

import jax
import jax.numpy as jnp
from jax import lax
from jax.experimental import pallas as pl
from jax.experimental.pallas import tpu as pltpu

_N = 10000
_E = 320000
_D = 128
_G = 128
_NL = 3
_EPS = 1e-16

_NODE_BLK = 2000
_EDGE_BLK = 4000


def _dense_body(u_ref, s_ref, v_ref, wq_ref, bq_ref, wk_ref, bk_ref,
                wv_ref, bv_ref, ws_ref, bs_ref,
                q_ref, k_ref, vv_ref, sk_ref, *, relu):
    h = u_ref[...] / (s_ref[...] + _EPS) + v_ref[...]
    if relu:
        h = jnp.maximum(h, 0.0)
    q_ref[...] = jnp.dot(h, wq_ref[...], preferred_element_type=jnp.float32) + bq_ref[...]
    k_ref[...] = jnp.dot(h, wk_ref[...], preferred_element_type=jnp.float32) + bk_ref[...]
    vv_ref[...] = jnp.dot(h, wv_ref[...], preferred_element_type=jnp.float32) + bv_ref[...]
    sk_ref[...] = jnp.dot(h, ws_ref[...], preferred_element_type=jnp.float32) + bs_ref[...]


def _dense(u, s, v, wq, bq, wk, bk, wv, bv, ws, bs, relu):
    n_spec = pl.BlockSpec((_NODE_BLK, _D), lambda i: (i, 0))
    s_spec = pl.BlockSpec((_NODE_BLK, 1), lambda i: (i, 0))
    w_spec = pl.BlockSpec((_D, _D), lambda i: (0, 0))
    b_spec = pl.BlockSpec((1, _D), lambda i: (0, 0))
    out = jax.ShapeDtypeStruct((_N, _D), jnp.float32)
    return pl.pallas_call(
        functools.partial(_dense_body, relu=relu),
        grid=(_N // _NODE_BLK,),
        in_specs=[n_spec, s_spec, n_spec,
                  w_spec, b_spec, w_spec, b_spec, w_spec, b_spec, w_spec, b_spec],
        out_specs=[n_spec, n_spec, n_spec, n_spec],
        out_shape=[out, out, out, out],
    )(u, s, v, wq, bq.reshape(1, _D), wk, bk.reshape(1, _D),
      wv, bv.reshape(1, _D), ws, bs.reshape(1, _D))


def _edge1_body(qd_ref, ks_ref, attr_ref, we_ref, be_ref, ar_ref):
    e = attr_ref[...] * we_ref[...] + be_ref[...]
    kj = ks_ref[...] + e
    ar_ref[...] = jnp.sum(qd_ref[...] * kj, axis=1, keepdims=True) * (
        1.0 / math.sqrt(float(_D)))


def _edge1(qd, ks, attr, we, be):
    e_spec = pl.BlockSpec((_EDGE_BLK, _D), lambda i: (i, 0))
    a_spec = pl.BlockSpec((_EDGE_BLK, 1), lambda i: (i, 0))
    w_spec = pl.BlockSpec((1, _D), lambda i: (0, 0))
    return pl.pallas_call(
        _edge1_body,
        grid=(_E // _EDGE_BLK,),
        in_specs=[e_spec, e_spec, a_spec, w_spec, w_spec],
        out_specs=a_spec,
        out_shape=jax.ShapeDtypeStruct((_E, 1), jnp.float32),
    )(qd, ks, attr, we.reshape(1, _D), be.reshape(1, _D))


def _edge2_body(vs_ref, attr_ref, ex_ref, we_ref, be_ref, msg_ref):
    e = attr_ref[...] * we_ref[...] + be_ref[...]
    msg_ref[...] = (vs_ref[...] + e) * ex_ref[...]


def _edge2(vs, attr, ex, we, be):
    e_spec = pl.BlockSpec((_EDGE_BLK, _D), lambda i: (i, 0))
    a_spec = pl.BlockSpec((_EDGE_BLK, 1), lambda i: (i, 0))
    w_spec = pl.BlockSpec((1, _D), lambda i: (0, 0))
    return pl.pallas_call(
        _edge2_body,
        grid=(_E // _EDGE_BLK,),
        in_specs=[e_spec, a_spec, a_spec, w_spec, w_spec],
        out_specs=e_spec,
        out_shape=jax.ShapeDtypeStruct((_E, _D), jnp.float32),
    )(vs, attr, ex, we.reshape(1, _D), be.reshape(1, _D))


def _head_body(agg_ref, s_ref, sk_ref, b_ref, energy_ref,
               wfc0_ref, bfc0_ref, wla_ref, wlb_ref, bl1_ref,
               wfc1_ref, bfc1_ref, wout_ref, bout_ref,
               out_ref, g_scr):
    i = pl.program_id(0)

    @pl.when(i == 0)
    def _():
        g_scr[...] = jnp.zeros_like(g_scr)

    h = jnp.maximum(agg_ref[...] / (s_ref[...] + _EPS) + sk_ref[...], 0.0)
    gid = lax.broadcasted_iota(jnp.int32, (_NODE_BLK, _G), 1)
    onehot = (b_ref[...] == gid).astype(jnp.float32)
    g_scr[...] += lax.dot_general(
        onehot, h, (((0,), (0,)), ((), ())),
        preferred_element_type=jnp.float32)

    @pl.when(i == pl.num_programs(0) - 1)
    def _():
        g = g_scr[...]
        y = jnp.maximum(
            jnp.dot(g, wfc0_ref[...], preferred_element_type=jnp.float32)
            + bfc0_ref[...], 0.0)
        z = jnp.maximum(
            jnp.dot(y, wla_ref[...], preferred_element_type=jnp.float32)
            + energy_ref[...] * wlb_ref[...] + bl1_ref[...], 0.0)
        z = z + y
        z = jnp.maximum(
            jnp.dot(z, wfc1_ref[...], preferred_element_type=jnp.float32)
            + bfc1_ref[...], 0.0)
        z = jnp.dot(z, wout_ref[...], preferred_element_type=jnp.float32) + bout_ref[...]
        m = jnp.max(z, axis=1, keepdims=True)
        out_ref[...] = z - (m + jnp.log(jnp.sum(jnp.exp(z - m), axis=1,
                                                keepdims=True)))


def _head(agg, s, sk, batchs, energy, wfc0, bfc0, wla, wlb, bl1,
          wfc1, bfc1, wout, bout):
    n_spec = pl.BlockSpec((_NODE_BLK, _D), lambda i: (i, 0))
    s_spec = pl.BlockSpec((_NODE_BLK, 1), lambda i: (i, 0))
    full = lambda a, b: pl.BlockSpec((a, b), lambda i: (0, 0))
    return pl.pallas_call(
        _head_body,
        grid=(_N // _NODE_BLK,),
        in_specs=[n_spec, s_spec, n_spec,
                  pl.BlockSpec((_NODE_BLK, 1), lambda i: (i, 0)),
                  full(_G, 1),
                  full(_D, _D), full(1, _D), full(_D, _D), full(1, _D),
                  full(1, _D), full(_D, _D), full(1, _D), full(_D, 2),
                  full(1, 2)],
        out_specs=full(_G, 2),
        out_shape=jax.ShapeDtypeStruct((_G, 2), jnp.float32),
        scratch_shapes=[pltpu.VMEM((_G, _D), jnp.float32)],
    )(agg, s, sk, batchs.reshape(_N, 1).astype(jnp.int32), energy,
      wfc0, bfc0.reshape(1, _D), wla, wlb.reshape(1, _D), bl1.reshape(1, _D),
      wfc1, bfc1.reshape(1, _D), wout, bout.reshape(1, 2))


def kernel(x, edge_index, edge_attr, batchs, energy, Wq, bq, Wk, bk, Wv, bv,
           We, be, Ws, bs, W_fc0, b_fc0, W_lin1, b_lin1, W_fc1, b_fc1,
           W_out, b_out):
    src = edge_index[0]
    dst = edge_index[1]

    u = x
    s = jnp.ones((_N, 1), jnp.float32)
    v_in = jnp.zeros((_N, _D), jnp.float32)

    for i in range(_NL):
        q, k, v, skip = _dense(u, s, v_in, Wq[i], bq[i], Wk[i], bk[i],
                               Wv[i], bv[i], Ws[i], bs[i], relu=(i > 0))
        qd = jnp.take(q, dst, axis=0)
        ks = jnp.take(k, src, axis=0)
        vs = jnp.take(v, src, axis=0)
        ar = _edge1(qd, ks, edge_attr, We[i], be[i])
        m = jnp.max(ar)
        ex = jnp.exp(ar - m)
        s_node = jax.ops.segment_sum(ex[:, 0], dst, num_segments=_N)
        msg = _edge2(vs, edge_attr, ex, We[i], be[i])
        agg = jax.ops.segment_sum(msg, dst, num_segments=_N)
        u = agg
        s = s_node.reshape(_N, 1)
        v_in = skip

    return _head(u, s, v_in, batchs, energy, W_fc0, b_fc0,
                 W_lin1[:_D], W_lin1[_D:_D + 1], b_lin1,
                 W_fc1, b_fc1, W_out, b_out)


# speedup vs baseline: 2.1331x; 1.0014x over previous
import functools
import math

import jax
import jax.numpy as jnp
from jax import lax
from jax.experimental import pallas as pl
from jax.experimental.pallas import tpu as pltpu

_N = 10000
_E = 320000
_D = 128
_G = 128
_NL = 3
_EPS = 1e-16

_NODE_BLK = 2000
_EDGE_BLK = 8000


def _dense_body(u_ref, s_ref, v_ref, wq_ref, bq_ref, wk_ref, bk_ref,
                wv_ref, bv_ref, ws_ref, bs_ref,
                q_ref, k_ref, vv_ref, sk_ref, *, relu):
    h = u_ref[...] / (s_ref[...] + _EPS) + v_ref[...]
    if relu:
        h = jnp.maximum(h, 0.0)
    q_ref[...] = jnp.dot(h, wq_ref[...], preferred_element_type=jnp.float32) + bq_ref[...]
    k_ref[...] = jnp.dot(h, wk_ref[...], preferred_element_type=jnp.float32) + bk_ref[...]
    vv_ref[...] = jnp.dot(h, wv_ref[...], preferred_element_type=jnp.float32) + bv_ref[...]
    sk_ref[...] = jnp.dot(h, ws_ref[...], preferred_element_type=jnp.float32) + bs_ref[...]


def _dense(u, s, v, wq, bq, wk, bk, wv, bv, ws, bs, relu):
    n_spec = pl.BlockSpec((_NODE_BLK, _D), lambda i: (i, 0))
    s_spec = pl.BlockSpec((_NODE_BLK, 1), lambda i: (i, 0))
    w_spec = pl.BlockSpec((_D, _D), lambda i: (0, 0))
    b_spec = pl.BlockSpec((1, _D), lambda i: (0, 0))
    out = jax.ShapeDtypeStruct((_N, _D), jnp.float32)
    return pl.pallas_call(
        functools.partial(_dense_body, relu=relu),
        grid=(_N // _NODE_BLK,),
        in_specs=[n_spec, s_spec, n_spec,
                  w_spec, b_spec, w_spec, b_spec, w_spec, b_spec, w_spec, b_spec],
        out_specs=[n_spec, n_spec, n_spec, n_spec],
        out_shape=[out, out, out, out],
    )(u, s, v, wq, bq.reshape(1, _D), wk, bk.reshape(1, _D),
      wv, bv.reshape(1, _D), ws, bs.reshape(1, _D))


def _edge1_body(qd_ref, ks_ref, attr_ref, we_ref, be_ref, ar_ref):
    e = attr_ref[...] * we_ref[...] + be_ref[...]
    kj = ks_ref[...] + e
    ar_ref[...] = jnp.sum(qd_ref[...] * kj, axis=1, keepdims=True) * (
        1.0 / math.sqrt(float(_D)))


def _edge1(qd, ks, attr, we, be):
    e_spec = pl.BlockSpec((_EDGE_BLK, _D), lambda i: (i, 0))
    a_spec = pl.BlockSpec((_EDGE_BLK, 1), lambda i: (i, 0))
    w_spec = pl.BlockSpec((1, _D), lambda i: (0, 0))
    return pl.pallas_call(
        _edge1_body,
        grid=(_E // _EDGE_BLK,),
        in_specs=[e_spec, e_spec, a_spec, w_spec, w_spec],
        out_specs=a_spec,
        out_shape=jax.ShapeDtypeStruct((_E, 1), jnp.float32),
    )(qd, ks, attr, we.reshape(1, _D), be.reshape(1, _D))


def _edge2_body(vs_ref, attr_ref, ex_ref, we_ref, be_ref, msg_ref):
    e = attr_ref[...] * we_ref[...] + be_ref[...]
    msg_ref[...] = (vs_ref[...] + e) * ex_ref[...]


def _edge2(vs, attr, ex, we, be):
    e_spec = pl.BlockSpec((_EDGE_BLK, _D), lambda i: (i, 0))
    a_spec = pl.BlockSpec((_EDGE_BLK, 1), lambda i: (i, 0))
    w_spec = pl.BlockSpec((1, _D), lambda i: (0, 0))
    return pl.pallas_call(
        _edge2_body,
        grid=(_E // _EDGE_BLK,),
        in_specs=[e_spec, a_spec, a_spec, w_spec, w_spec],
        out_specs=e_spec,
        out_shape=jax.ShapeDtypeStruct((_E, _D), jnp.float32),
    )(vs, attr, ex, we.reshape(1, _D), be.reshape(1, _D))


def _head_body(agg_ref, s_ref, sk_ref, b_ref, energy_ref,
               wfc0_ref, bfc0_ref, wla_ref, wlb_ref, bl1_ref,
               wfc1_ref, bfc1_ref, wout_ref, bout_ref,
               out_ref, g_scr):
    i = pl.program_id(0)

    @pl.when(i == 0)
    def _():
        g_scr[...] = jnp.zeros_like(g_scr)

    h = jnp.maximum(agg_ref[...] / (s_ref[...] + _EPS) + sk_ref[...], 0.0)
    gid = lax.broadcasted_iota(jnp.int32, (_NODE_BLK, _G), 1)
    onehot = (b_ref[...] == gid).astype(jnp.float32)
    g_scr[...] += lax.dot_general(
        onehot, h, (((0,), (0,)), ((), ())),
        preferred_element_type=jnp.float32)

    @pl.when(i == pl.num_programs(0) - 1)
    def _():
        g = g_scr[...]
        y = jnp.maximum(
            jnp.dot(g, wfc0_ref[...], preferred_element_type=jnp.float32)
            + bfc0_ref[...], 0.0)
        z = jnp.maximum(
            jnp.dot(y, wla_ref[...], preferred_element_type=jnp.float32)
            + energy_ref[...] * wlb_ref[...] + bl1_ref[...], 0.0)
        z = z + y
        z = jnp.maximum(
            jnp.dot(z, wfc1_ref[...], preferred_element_type=jnp.float32)
            + bfc1_ref[...], 0.0)
        z = jnp.dot(z, wout_ref[...], preferred_element_type=jnp.float32) + bout_ref[...]
        m = jnp.max(z, axis=1, keepdims=True)
        out_ref[...] = z - (m + jnp.log(jnp.sum(jnp.exp(z - m), axis=1,
                                                keepdims=True)))


def _head(agg, s, sk, batchs, energy, wfc0, bfc0, wla, wlb, bl1,
          wfc1, bfc1, wout, bout):
    n_spec = pl.BlockSpec((_NODE_BLK, _D), lambda i: (i, 0))
    s_spec = pl.BlockSpec((_NODE_BLK, 1), lambda i: (i, 0))
    full = lambda a, b: pl.BlockSpec((a, b), lambda i: (0, 0))
    return pl.pallas_call(
        _head_body,
        grid=(_N // _NODE_BLK,),
        in_specs=[n_spec, s_spec, n_spec,
                  pl.BlockSpec((_NODE_BLK, 1), lambda i: (i, 0)),
                  full(_G, 1),
                  full(_D, _D), full(1, _D), full(_D, _D), full(1, _D),
                  full(1, _D), full(_D, _D), full(1, _D), full(_D, 2),
                  full(1, 2)],
        out_specs=full(_G, 2),
        out_shape=jax.ShapeDtypeStruct((_G, 2), jnp.float32),
        scratch_shapes=[pltpu.VMEM((_G, _D), jnp.float32)],
    )(agg, s, sk, batchs.reshape(_N, 1).astype(jnp.int32), energy,
      wfc0, bfc0.reshape(1, _D), wla, wlb.reshape(1, _D), bl1.reshape(1, _D),
      wfc1, bfc1.reshape(1, _D), wout, bout.reshape(1, 2))


def kernel(x, edge_index, edge_attr, batchs, energy, Wq, bq, Wk, bk, Wv, bv,
           We, be, Ws, bs, W_fc0, b_fc0, W_lin1, b_lin1, W_fc1, b_fc1,
           W_out, b_out):
    src = edge_index[0]
    dst = edge_index[1]

    u = x
    s = jnp.ones((_N, 1), jnp.float32)
    v_in = jnp.zeros((_N, _D), jnp.float32)

    for i in range(_NL):
        q, k, v, skip = _dense(u, s, v_in, Wq[i], bq[i], Wk[i], bk[i],
                               Wv[i], bv[i], Ws[i], bs[i], relu=(i > 0))
        qd = jnp.take(q, dst, axis=0)
        ks = jnp.take(k, src, axis=0)
        vs = jnp.take(v, src, axis=0)
        ar = _edge1(qd, ks, edge_attr, We[i], be[i])
        m = jnp.max(ar)
        ex = jnp.exp(ar - m)
        s_node = jax.ops.segment_sum(ex[:, 0], dst, num_segments=_N)
        msg = _edge2(vs, edge_attr, ex, We[i], be[i])
        agg = jax.ops.segment_sum(msg, dst, num_segments=_N)
        u = agg
        s = s_node.reshape(_N, 1)
        v_in = skip

    return _head(u, s, v_in, batchs, energy, W_fc0, b_fc0,
                 W_lin1[:_D], W_lin1[_D:_D + 1], b_lin1,
                 W_fc1, b_fc1, W_out, b_out)
